# baseline (speedup 1.0000x reference)
import functools

import jax
import jax.numpy as jnp
from jax.experimental import pallas as pl
from jax.experimental.pallas import tpu as pltpu

N_CODES = 8192
CODE_DIM = 256
COMMITMENT_COST = 0.25

M_BLK = 1024
K_BLK = 2048


def _argmin_body(zsq_ref, wsq_ref, z2b_ref, wb_ref, idx_out,
                 minval_s, minidx_s):
    k = pl.program_id(1)
    nk = pl.num_programs(1)

    mm = jax.lax.dot_general(
        z2b_ref[...], wb_ref[...],
        dimension_numbers=(((1,), (1,)), ((), ())),
        preferred_element_type=jnp.float32)
    d = (zsq_ref[...] + wsq_ref[...]) - mm

    lm = jnp.min(d, axis=1, keepdims=True)
    col = jax.lax.broadcasted_iota(jnp.int32, d.shape, 1) + k * K_BLK
    big = jnp.int32(jnp.iinfo(jnp.int32).max)
    li = jnp.min(jnp.where(d == lm, col, big), axis=1, keepdims=True)

    @pl.when(k == 0)
    def _():
        minval_s[...] = lm.astype(jnp.bfloat16).astype(jnp.float32)
        minidx_s[...] = li

    @pl.when(k > 0)
    def _():
        m = minval_s[...]
        i = minidx_s[...]
        better = (lm < m) | ((lm == m) & (li < i))
        minidx_s[...] = jnp.where(better, li, i)
        minval_s[...] = jnp.where(better, lm, m).astype(
            jnp.bfloat16).astype(jnp.float32)

    @pl.when(k == nk - 1)
    def _():
        idx_out[...] = minidx_s[...]


def _argmin_call(z2b, wb, zsq, wsq):
    M = z2b.shape[0]
    grid = (M // M_BLK, N_CODES // K_BLK)
    return pl.pallas_call(
        _argmin_body,
        grid=grid,
        in_specs=[
            pl.BlockSpec((M_BLK, 1), lambda m, k: (m, 0)),
            pl.BlockSpec((1, K_BLK), lambda m, k: (0, k)),
            pl.BlockSpec((M_BLK, CODE_DIM), lambda m, k: (m, 0)),
            pl.BlockSpec((K_BLK, CODE_DIM), lambda m, k: (k, 0)),
        ],
        out_specs=pl.BlockSpec((M_BLK, 1), lambda m, k: (m, 0)),
        out_shape=jax.ShapeDtypeStruct((M, 1), jnp.int32),
        scratch_shapes=[
            pltpu.VMEM((M_BLK, 1), jnp.float32),
            pltpu.VMEM((M_BLK, 1), jnp.int32),
        ],
    )(zsq, wsq, z2b, wb)


def kernel(z, W):
    B, N, D = z.shape
    z_flat = z.reshape(-1, D)
    zsq = jnp.sum(z_flat ** 2, axis=1, keepdims=True)
    wsq = jnp.sum(W ** 2, axis=1)[None, :]
    z2b = (2.0 * z_flat).astype(jnp.bfloat16)
    wb = W.astype(jnp.bfloat16)

    idx2 = _argmin_call(z2b, wb, zsq, wsq)
    indices_flat = idx2[:, 0]
    indices = indices_flat.reshape(B, N)

    z_q_flat = jnp.take(W, indices_flat, axis=0)
    z_q = z_q_flat.reshape(B, N, D)
    commitment_loss = jnp.mean((z - jax.lax.stop_gradient(z_q)) ** 2)
    codebook_loss = jnp.mean((z_q - jax.lax.stop_gradient(z)) ** 2)
    vq_loss = commitment_loss * COMMITMENT_COST + codebook_loss
    z_q_st = z + jax.lax.stop_gradient(z_q - z)
    counts = jnp.bincount(indices_flat, length=N_CODES).astype(jnp.float32)
    avg_probs = counts / indices_flat.shape[0]
    perplexity = jnp.exp(-jnp.sum(avg_probs * jnp.log(avg_probs + 1e-10)))
    return (z_q_st, indices, vq_loss, perplexity)
